# baseline (speedup 1.0000x reference)
import functools

import jax
import jax.numpy as jnp
from jax import lax
from jax.experimental import pallas as pl
from jax.experimental.pallas import tpu as pltpu
from jax.experimental.pallas import tpu_sc as plsc

NC = 2
NS = 16
NW = NC * NS
L = 16

C = 128
NP = 10112
CLAMP = NP - 8
BIG = 1 << 30

_DCH = [(0, C), (C, C), (2 * C, C), (3 * C, C), (4 * C, 120)]


def _mesh():
    return plsc.VectorSubcoreMesh(core_axis_name="c", subcore_axis_name="s",
                                  num_cores=NC, num_subcores=NS)


def _phase_sums(N, D, E_pad):
    db = D // L
    nch = E_pad // (NW * C)
    epw = nch * C

    @functools.partial(
        pl.kernel,
        mesh=_mesh(),
        out_type=jax.ShapeDtypeStruct((NC * NP, D), jnp.float32),
        compiler_params=pltpu.CompilerParams(needs_layout_passes=False),
        scratch_types=[
            pltpu.VMEM((C,), jnp.int32),
            pltpu.VMEM((C,), jnp.int32),
            pltpu.VMEM((C, D), jnp.float32),
            pltpu.VMEM_SHARED((NP, D), jnp.float32),
            pltpu.SemaphoreType.DMA,
        ],
    )
    def k(feat_hbm, nbr_hbm, seg_hbm, psum_hbm,
          idx_v, seg_v, rows_v, acc_sh, sem):
        cid = lax.axis_index("c")
        sid = lax.axis_index("s")
        w = sid * NC + cid

        zeros16 = jnp.zeros((L,), jnp.float32)

        @pl.loop(0, C)
        def _(r):
            for j in range(db):
                rows_v[r, pl.ds(j * L, L)] = zeros16

        bz = pl.multiple_of(sid * (NP // NS), 8)
        for off, n in _DCH:
            pltpu.sync_copy(rows_v.at[pl.ds(0, n)], acc_sh.at[pl.ds(bz + off, n)])
        plsc.subcore_barrier()

        clamp16 = jnp.full((L,), CLAMP, jnp.int32)

        @pl.loop(0, nch)
        def _(i):
            b = pl.multiple_of(w * epw + i * C, C)
            pltpu.sync_copy(nbr_hbm.at[pl.ds(b, C)], idx_v)
            pltpu.sync_copy(seg_hbm.at[pl.ds(b, C)], seg_v)
            for j in range(C // L):
                s = pl.ds(j * L, L)
                seg_v[s] = jnp.minimum(seg_v[s], clamp16)
            pltpu.async_copy(feat_hbm.at[idx_v], rows_v, sem).wait()
            pltpu.sync_copy(rows_v, acc_sh.at[seg_v], add=True)

        plsc.subcore_barrier()

        bh = pl.multiple_of(cid * NP + sid * (NP // NS), 8)
        for off, n in _DCH:
            pltpu.sync_copy(acc_sh.at[pl.ds(bz + off, n)], rows_v.at[pl.ds(0, n)])
            pltpu.sync_copy(rows_v.at[pl.ds(0, n)], psum_hbm.at[pl.ds(bh + off, n)])

    return k


def _phase_counts(D, E_pad):
    db = D // L
    nch = E_pad // (NW * C)

    @functools.partial(
        pl.kernel,
        mesh=_mesh(),
        out_type=jax.ShapeDtypeStruct((NC * NP, D), jnp.float32),
        compiler_params=pltpu.CompilerParams(needs_layout_passes=False),
        scratch_types=[
            pltpu.VMEM((nch, C), jnp.int32),
            pltpu.VMEM((C, D), jnp.float32),
            pltpu.VMEM_SHARED((NP, D), jnp.float32),
        ],
    )
    def k(seg_hbm, pcnt_hbm, seg_s, buf_v, cnt_sh):
        cid = lax.axis_index("c")
        sid = lax.axis_index("s")
        w = sid * NC + cid

        bs = pl.multiple_of(w * nch, 8)
        pltpu.sync_copy(seg_hbm.at[pl.ds(bs, nch)], seg_s)

        zeros16 = jnp.zeros((L,), jnp.float32)
        ones16 = jnp.ones((L,), jnp.float32)

        @pl.loop(0, C)
        def _(r):
            for j in range(db):
                buf_v[r, pl.ds(j * L, L)] = zeros16

        bz = pl.multiple_of(sid * (NP // NS), 8)
        for off, n in _DCH:
            pltpu.sync_copy(buf_v.at[pl.ds(0, n)], cnt_sh.at[pl.ds(bz + off, n)])
        plsc.subcore_barrier()

        @pl.loop(0, C)
        def _(r):
            buf_v[r, pl.ds(0, L)] = ones16

        @pl.loop(0, nch)
        def _(i):
            pltpu.sync_copy(buf_v, cnt_sh.at[seg_s.at[i]], add=True)

        plsc.subcore_barrier()

        bh = pl.multiple_of(cid * NP + sid * (NP // NS), 8)
        for off, n in _DCH:
            pltpu.sync_copy(cnt_sh.at[pl.ds(bz + off, n)], buf_v.at[pl.ds(0, n)])
            pltpu.sync_copy(buf_v.at[pl.ds(0, n)], pcnt_hbm.at[pl.ds(bh + off, n)])

    return k


def _phase_merge(D):
    db = D // L
    nchunk = NP // C

    @functools.partial(
        pl.kernel,
        mesh=_mesh(),
        out_type=jax.ShapeDtypeStruct((NP, D), jnp.float32),
        compiler_params=pltpu.CompilerParams(needs_layout_passes=False),
        scratch_types=[
            pltpu.VMEM((C, D), jnp.float32),
            pltpu.VMEM((C, D), jnp.float32),
            pltpu.VMEM((C, D), jnp.float32),
            pltpu.VMEM((C, D), jnp.float32),
        ],
    )
    def k(psum_hbm, pcnt_hbm, out_hbm, p0_v, p1_v, c0_v, c1_v):
        cid = lax.axis_index("c")
        sid = lax.axis_index("s")
        w = sid * NC + cid

        for j in range((nchunk + NW - 1) // NW):
            ch = w + j * NW

            @pl.when(ch < nchunk)
            def _():
                base = pl.multiple_of(ch * C, 8)
                pltpu.sync_copy(psum_hbm.at[pl.ds(base, C)], p0_v)
                pltpu.sync_copy(psum_hbm.at[pl.ds(NP + base, C)], p1_v)
                pltpu.sync_copy(pcnt_hbm.at[pl.ds(base, C)], c0_v)
                pltpu.sync_copy(pcnt_hbm.at[pl.ds(NP + base, C)], c1_v)

                @pl.loop(0, C)
                def _(r):
                    cv = c0_v[r, pl.ds(0, L)] + c1_v[r, pl.ds(0, L)]
                    inv = jnp.where(cv > 0.0, 1.0 / jnp.maximum(cv, 1.0), 0.0)
                    for jj in range(db):
                        s = pl.ds(jj * L, L)
                        p0_v[r, s] = (p0_v[r, s] + p1_v[r, s]) * inv

                pltpu.sync_copy(p0_v, out_hbm.at[pl.ds(base, C)])

    return k


def kernel(features, neighbor_idx, segment_ids, num_neighbors):
    N, D = features.shape
    E = neighbor_idx.shape[0]
    grain = NW * C * 8
    E_pad = -(-E // grain) * grain
    pad = E_pad - E

    nbr_p = jnp.concatenate([neighbor_idx, jnp.zeros((pad,), jnp.int32)])
    seg_p = jnp.concatenate([segment_ids, jnp.full((pad,), CLAMP, jnp.int32)])
    seg2 = seg_p.reshape(E_pad // C, C)

    psum = _phase_sums(N, D, E_pad)(features, nbr_p, seg_p)
    pcnt = _phase_counts(D, E_pad)(seg2)
    out = _phase_merge(D)(psum, pcnt)
    return out[:N]


# speedup vs baseline: 1.3228x; 1.3228x over previous
import functools

import jax
import jax.numpy as jnp
from jax import lax
from jax.experimental import pallas as pl
from jax.experimental.pallas import tpu as pltpu
from jax.experimental.pallas import tpu_sc as plsc

NC = 2
NS = 16
NW = NC * NS
L = 16

C = 128
NP = 10112
CLAMP = NP - 8
BIG = 1 << 30

_DCH = [(0, C), (C, C), (2 * C, C), (3 * C, C), (4 * C, 120)]


def _mesh():
    return plsc.VectorSubcoreMesh(core_axis_name="c", subcore_axis_name="s",
                                  num_cores=NC, num_subcores=NS)


def _phase_sums(N, D, E_pad):
    db = D // L
    nch = E_pad // (NW * C)
    epw = nch * C

    @functools.partial(
        pl.kernel,
        mesh=_mesh(),
        out_type=jax.ShapeDtypeStruct((NC * NP, D), jnp.float32),
        compiler_params=pltpu.CompilerParams(needs_layout_passes=False),
        scratch_types=[
            pltpu.VMEM((C,), jnp.int32),
            pltpu.VMEM((C,), jnp.int32),
            pltpu.VMEM((C, D), jnp.float32),
            pltpu.VMEM_SHARED((NP, D), jnp.float32),
            pltpu.SemaphoreType.DMA,
        ],
    )
    def k(feat_hbm, nbr_hbm, seg_hbm, psum_hbm,
          idx_v, seg_v, rows_v, acc_sh, sem):
        cid = lax.axis_index("c")
        sid = lax.axis_index("s")
        w = sid * NC + cid

        zeros16 = jnp.zeros((L,), jnp.float32)

        @pl.loop(0, C)
        def _(r):
            for j in range(db):
                rows_v[r, pl.ds(j * L, L)] = zeros16

        bz = pl.multiple_of(sid * (NP // NS), 8)
        for off, n in _DCH:
            pltpu.sync_copy(rows_v.at[pl.ds(0, n)], acc_sh.at[pl.ds(bz + off, n)])
        plsc.subcore_barrier()

        clamp16 = jnp.full((L,), CLAMP, jnp.int32)

        @pl.loop(0, nch)
        def _(i):
            b = pl.multiple_of(w * epw + i * C, C)
            pltpu.sync_copy(nbr_hbm.at[pl.ds(b, C)], idx_v)
            pltpu.sync_copy(seg_hbm.at[pl.ds(b, C)], seg_v)
            for j in range(C // L):
                s = pl.ds(j * L, L)
                seg_v[s] = jnp.minimum(seg_v[s], clamp16)
            pltpu.async_copy(feat_hbm.at[idx_v], rows_v, sem).wait()
            pltpu.sync_copy(rows_v, acc_sh.at[seg_v], add=True)

        plsc.subcore_barrier()

        bh = pl.multiple_of(cid * NP + sid * (NP // NS), 8)
        for off, n in _DCH:
            pltpu.sync_copy(acc_sh.at[pl.ds(bz + off, n)], rows_v.at[pl.ds(0, n)])
            pltpu.sync_copy(rows_v.at[pl.ds(0, n)], psum_hbm.at[pl.ds(bh + off, n)])

    return k


def _phase_counts(D, E_pad):
    db = D // L
    nch = E_pad // (NW * C)
    epw = nch * C

    @functools.partial(
        pl.kernel,
        mesh=_mesh(),
        out_type=jax.ShapeDtypeStruct((NC * NP, D), jnp.float32),
        compiler_params=pltpu.CompilerParams(needs_layout_passes=False),
        scratch_types=[
            pltpu.VMEM((C,), jnp.int32),
            pltpu.VMEM((C, D), jnp.float32),
            pltpu.VMEM_SHARED((NP, D), jnp.float32),
        ],
    )
    def k(seg_hbm, pcnt_hbm, seg_v, buf_v, cnt_sh):
        cid = lax.axis_index("c")
        sid = lax.axis_index("s")
        w = sid * NC + cid

        zeros16 = jnp.zeros((L,), jnp.float32)
        ones16 = jnp.ones((L,), jnp.float32)

        @pl.loop(0, C)
        def _(r):
            for j in range(db):
                buf_v[r, pl.ds(j * L, L)] = zeros16

        bz = pl.multiple_of(sid * (NP // NS), 8)
        for off, n in _DCH:
            pltpu.sync_copy(buf_v.at[pl.ds(0, n)], cnt_sh.at[pl.ds(bz + off, n)])
        plsc.subcore_barrier()

        @pl.loop(0, C)
        def _(r):
            buf_v[r, pl.ds(0, L)] = ones16

        clamp16 = jnp.full((L,), CLAMP, jnp.int32)

        @pl.loop(0, nch)
        def _(i):
            b = pl.multiple_of(w * epw + i * C, C)
            pltpu.sync_copy(seg_hbm.at[pl.ds(b, C)], seg_v)
            for j in range(C // L):
                s = pl.ds(j * L, L)
                seg_v[s] = jnp.minimum(seg_v[s], clamp16)
            pltpu.sync_copy(buf_v, cnt_sh.at[seg_v], add=True)

        plsc.subcore_barrier()

        bh = pl.multiple_of(cid * NP + sid * (NP // NS), 8)
        for off, n in _DCH:
            pltpu.sync_copy(cnt_sh.at[pl.ds(bz + off, n)], buf_v.at[pl.ds(0, n)])
            pltpu.sync_copy(buf_v.at[pl.ds(0, n)], pcnt_hbm.at[pl.ds(bh + off, n)])

    return k


def _phase_merge(D):
    db = D // L
    nchunk = NP // C

    @functools.partial(
        pl.kernel,
        mesh=_mesh(),
        out_type=jax.ShapeDtypeStruct((NP, D), jnp.float32),
        compiler_params=pltpu.CompilerParams(needs_layout_passes=False),
        scratch_types=[
            pltpu.VMEM((C, D), jnp.float32),
            pltpu.VMEM((C, D), jnp.float32),
            pltpu.VMEM((C, D), jnp.float32),
            pltpu.VMEM((C, D), jnp.float32),
        ],
    )
    def k(psum_hbm, pcnt_hbm, out_hbm, p0_v, p1_v, c0_v, c1_v):
        cid = lax.axis_index("c")
        sid = lax.axis_index("s")
        w = sid * NC + cid

        for j in range((nchunk + NW - 1) // NW):
            ch = w + j * NW

            @pl.when(ch < nchunk)
            def _():
                base = pl.multiple_of(ch * C, 8)
                pltpu.sync_copy(psum_hbm.at[pl.ds(base, C)], p0_v)
                pltpu.sync_copy(psum_hbm.at[pl.ds(NP + base, C)], p1_v)
                pltpu.sync_copy(pcnt_hbm.at[pl.ds(base, C)], c0_v)
                pltpu.sync_copy(pcnt_hbm.at[pl.ds(NP + base, C)], c1_v)

                @pl.loop(0, C)
                def _(r):
                    cv = c0_v[r, pl.ds(0, L)] + c1_v[r, pl.ds(0, L)]
                    inv = jnp.where(cv > 0.0, 1.0 / jnp.maximum(cv, 1.0), 0.0)
                    for jj in range(db):
                        s = pl.ds(jj * L, L)
                        p0_v[r, s] = (p0_v[r, s] + p1_v[r, s]) * inv

                pltpu.sync_copy(p0_v, out_hbm.at[pl.ds(base, C)])

    return k


def kernel(features, neighbor_idx, segment_ids, num_neighbors):
    N, D = features.shape
    E = neighbor_idx.shape[0]
    E_pad = -(-E // (NW * C)) * (NW * C)
    pad = E_pad - E

    nbr_p = jnp.concatenate([neighbor_idx, jnp.zeros((pad,), jnp.int32)])
    seg_p = jnp.concatenate([segment_ids, jnp.full((pad,), BIG, jnp.int32)])

    psum = _phase_sums(N, D, E_pad)(features, nbr_p, seg_p)
    pcnt = _phase_counts(D, E_pad)(seg_p)
    out = _phase_merge(D)(psum, pcnt)
    return out[:N]


# speedup vs baseline: 1.4002x; 1.0585x over previous
import functools

import jax
import jax.numpy as jnp
from jax import lax
from jax.experimental import pallas as pl
from jax.experimental.pallas import tpu as pltpu
from jax.experimental.pallas import tpu_sc as plsc

NC = 2
NS = 16
NW = NC * NS
L = 16

C = 128
NP = 10112
CLAMP = NP - 8
BIG = 1 << 30

_DCH = [(0, C), (C, C), (2 * C, C), (3 * C, C), (4 * C, 120)]


def _mesh():
    return plsc.VectorSubcoreMesh(core_axis_name="c", subcore_axis_name="s",
                                  num_cores=NC, num_subcores=NS)


def _phase_sums(N, D, E_pad):
    db = D // L
    nch = E_pad // (NW * C)
    epw = nch * C

    @functools.partial(
        pl.kernel,
        mesh=_mesh(),
        out_type=jax.ShapeDtypeStruct((NC * NP, D), jnp.float32),
        compiler_params=pltpu.CompilerParams(needs_layout_passes=False),
        scratch_types=[
            pltpu.VMEM((C,), jnp.int32),
            pltpu.VMEM((C,), jnp.int32),
            pltpu.VMEM((C, D), jnp.float32),
            pltpu.VMEM_SHARED((NP, D), jnp.float32),
            pltpu.SemaphoreType.DMA,
        ],
    )
    def k(feat_hbm, nbr_hbm, seg_hbm, psum_hbm,
          idx_v, seg_v, rows_v, acc_sh, sem):
        cid = lax.axis_index("c")
        sid = lax.axis_index("s")
        w = sid * NC + cid

        zeros16 = jnp.zeros((L,), jnp.float32)

        @pl.loop(0, C)
        def _(r):
            for j in range(db):
                rows_v[r, pl.ds(j * L, L)] = zeros16

        bz = pl.multiple_of(sid * (NP // NS), 8)
        for off, n in _DCH:
            pltpu.sync_copy(rows_v.at[pl.ds(0, n)], acc_sh.at[pl.ds(bz + off, n)])
        plsc.subcore_barrier()

        clamp16 = jnp.full((L,), CLAMP, jnp.int32)

        @pl.loop(0, nch)
        def _(i):
            b = pl.multiple_of(w * epw + i * C, C)
            pltpu.async_copy(nbr_hbm.at[pl.ds(b, C)], idx_v, sem)
            pltpu.async_copy(seg_hbm.at[pl.ds(b, C)], seg_v, sem)
            pltpu.make_async_copy(nbr_hbm.at[pl.ds(b, C)], idx_v, sem).wait()
            pltpu.make_async_copy(seg_hbm.at[pl.ds(b, C)], seg_v, sem).wait()
            for j in range(C // L):
                s = pl.ds(j * L, L)
                seg_v[s] = jnp.minimum(seg_v[s], clamp16)
            pltpu.async_copy(feat_hbm.at[idx_v], rows_v, sem).wait()
            pltpu.sync_copy(rows_v, acc_sh.at[seg_v], add=True)

        plsc.subcore_barrier()

        bh = pl.multiple_of(cid * NP + sid * (NP // NS), 8)
        for off, n in _DCH:
            pltpu.sync_copy(acc_sh.at[pl.ds(bz + off, n)], rows_v.at[pl.ds(0, n)])
            pltpu.sync_copy(rows_v.at[pl.ds(0, n)], psum_hbm.at[pl.ds(bh + off, n)])

    return k


def _phase_counts(D, E_pad):
    db = D // L
    nch = E_pad // (NW * C)
    epw = nch * C

    @functools.partial(
        pl.kernel,
        mesh=_mesh(),
        out_type=jax.ShapeDtypeStruct((NC * NP, D), jnp.float32),
        compiler_params=pltpu.CompilerParams(needs_layout_passes=False),
        scratch_types=[
            pltpu.VMEM((C,), jnp.int32),
            pltpu.VMEM((C, D), jnp.float32),
            pltpu.VMEM_SHARED((NP, D), jnp.float32),
        ],
    )
    def k(seg_hbm, pcnt_hbm, seg_v, buf_v, cnt_sh):
        cid = lax.axis_index("c")
        sid = lax.axis_index("s")
        w = sid * NC + cid

        zeros16 = jnp.zeros((L,), jnp.float32)
        ones16 = jnp.ones((L,), jnp.float32)

        @pl.loop(0, C)
        def _(r):
            for j in range(db):
                buf_v[r, pl.ds(j * L, L)] = zeros16

        bz = pl.multiple_of(sid * (NP // NS), 8)
        for off, n in _DCH:
            pltpu.sync_copy(buf_v.at[pl.ds(0, n)], cnt_sh.at[pl.ds(bz + off, n)])
        plsc.subcore_barrier()

        @pl.loop(0, C)
        def _(r):
            buf_v[r, pl.ds(0, L)] = ones16

        clamp16 = jnp.full((L,), CLAMP, jnp.int32)

        @pl.loop(0, nch)
        def _(i):
            b = pl.multiple_of(w * epw + i * C, C)
            pltpu.sync_copy(seg_hbm.at[pl.ds(b, C)], seg_v)
            for j in range(C // L):
                s = pl.ds(j * L, L)
                seg_v[s] = jnp.minimum(seg_v[s], clamp16)
            pltpu.sync_copy(buf_v, cnt_sh.at[seg_v], add=True)

        plsc.subcore_barrier()

        bh = pl.multiple_of(cid * NP + sid * (NP // NS), 8)
        for off, n in _DCH:
            pltpu.sync_copy(cnt_sh.at[pl.ds(bz + off, n)], buf_v.at[pl.ds(0, n)])
            pltpu.sync_copy(buf_v.at[pl.ds(0, n)], pcnt_hbm.at[pl.ds(bh + off, n)])

    return k


def _phase_merge(D):
    db = D // L
    nchunk = NP // C

    @functools.partial(
        pl.kernel,
        mesh=_mesh(),
        out_type=jax.ShapeDtypeStruct((NP, D), jnp.float32),
        compiler_params=pltpu.CompilerParams(needs_layout_passes=False),
        scratch_types=[
            pltpu.VMEM((C, D), jnp.float32),
            pltpu.VMEM((C, D), jnp.float32),
            pltpu.VMEM((C, D), jnp.float32),
            pltpu.VMEM((C, D), jnp.float32),
        ],
    )
    def k(psum_hbm, pcnt_hbm, out_hbm, p0_v, p1_v, c0_v, c1_v):
        cid = lax.axis_index("c")
        sid = lax.axis_index("s")
        w = sid * NC + cid

        for j in range((nchunk + NW - 1) // NW):
            ch = w + j * NW

            @pl.when(ch < nchunk)
            def _():
                base = pl.multiple_of(ch * C, 8)
                pltpu.sync_copy(psum_hbm.at[pl.ds(base, C)], p0_v)
                pltpu.sync_copy(psum_hbm.at[pl.ds(NP + base, C)], p1_v)
                pltpu.sync_copy(pcnt_hbm.at[pl.ds(base, C)], c0_v)
                pltpu.sync_copy(pcnt_hbm.at[pl.ds(NP + base, C)], c1_v)

                @pl.loop(0, C)
                def _(r):
                    cv = c0_v[r, pl.ds(0, L)] + c1_v[r, pl.ds(0, L)]
                    inv = jnp.where(cv > 0.0, 1.0 / jnp.maximum(cv, 1.0), 0.0)
                    for jj in range(db):
                        s = pl.ds(jj * L, L)
                        p0_v[r, s] = (p0_v[r, s] + p1_v[r, s]) * inv

                pltpu.sync_copy(p0_v, out_hbm.at[pl.ds(base, C)])

    return k


def kernel(features, neighbor_idx, segment_ids, num_neighbors):
    N, D = features.shape
    E = neighbor_idx.shape[0]
    E_pad = -(-E // (NW * C)) * (NW * C)
    pad = E_pad - E

    nbr_p = jnp.concatenate([neighbor_idx, jnp.zeros((pad,), jnp.int32)])
    seg_p = jnp.concatenate([segment_ids, jnp.full((pad,), BIG, jnp.int32)])

    psum = _phase_sums(N, D, E_pad)(features, nbr_p, seg_p)
    pcnt = _phase_counts(D, E_pad)(seg_p)
    out = _phase_merge(D)(psum, pcnt)
    return out[:N]


# speedup vs baseline: 1.4025x; 1.0017x over previous
import functools

import jax
import jax.numpy as jnp
from jax import lax
from jax.experimental import pallas as pl
from jax.experimental.pallas import tpu as pltpu
from jax.experimental.pallas import tpu_sc as plsc

NC = 2
NS = 16
NW = NC * NS
L = 16

C = 128
NP = 10112
CLAMP = NP - 8
BIG = 1 << 30

_DCH = [(0, C), (C, C), (2 * C, C), (3 * C, C), (4 * C, 120)]


def _mesh():
    return plsc.VectorSubcoreMesh(core_axis_name="c", subcore_axis_name="s",
                                  num_cores=NC, num_subcores=NS)


def _phase_sums(N, D, E_pad):
    db = D // L
    nch = E_pad // (NW * C)
    epw = nch * C

    @functools.partial(
        pl.kernel,
        mesh=_mesh(),
        out_type=jax.ShapeDtypeStruct((NC * NP, D), jnp.float32),
        compiler_params=pltpu.CompilerParams(needs_layout_passes=False),
        scratch_types=[
            pltpu.VMEM((C,), jnp.int32),
            pltpu.VMEM((C,), jnp.int32),
            pltpu.VMEM((C, D), jnp.float32),
            pltpu.VMEM_SHARED((NP, D), jnp.float32),
            pltpu.SemaphoreType.DMA,
            pltpu.SemaphoreType.DMA,
            pltpu.SemaphoreType.DMA,
        ],
    )
    def k(feat_hbm, nbr_hbm, seg_hbm, psum_hbm,
          idx_v, seg_v, rows_v, acc_sh, sem, sem2, semg):
        cid = lax.axis_index("c")
        sid = lax.axis_index("s")
        w = sid * NC + cid

        zeros16 = jnp.zeros((L,), jnp.float32)

        @pl.loop(0, C)
        def _(r):
            for j in range(db):
                rows_v[r, pl.ds(j * L, L)] = zeros16

        bz = pl.multiple_of(sid * (NP // NS), 8)
        for off, n in _DCH:
            pltpu.sync_copy(rows_v.at[pl.ds(0, n)], acc_sh.at[pl.ds(bz + off, n)])
        plsc.subcore_barrier()

        clamp16 = jnp.full((L,), CLAMP, jnp.int32)

        @pl.loop(0, nch)
        def _(i):
            b = pl.multiple_of(w * epw + i * C, C)
            pltpu.async_copy(nbr_hbm.at[pl.ds(b, C)], idx_v, sem)
            pltpu.async_copy(seg_hbm.at[pl.ds(b, C)], seg_v, sem2)
            pltpu.make_async_copy(nbr_hbm.at[pl.ds(b, C)], idx_v, sem).wait()
            pltpu.async_copy(feat_hbm.at[idx_v], rows_v, semg)
            pltpu.make_async_copy(seg_hbm.at[pl.ds(b, C)], seg_v, sem2).wait()
            for j in range(C // L):
                s = pl.ds(j * L, L)
                seg_v[s] = jnp.minimum(seg_v[s], clamp16)
            pltpu.make_async_copy(feat_hbm.at[idx_v], rows_v, semg).wait()
            pltpu.sync_copy(rows_v, acc_sh.at[seg_v], add=True)

        plsc.subcore_barrier()

        bh = pl.multiple_of(cid * NP + sid * (NP // NS), 8)
        for off, n in _DCH:
            pltpu.sync_copy(acc_sh.at[pl.ds(bz + off, n)], rows_v.at[pl.ds(0, n)])
            pltpu.sync_copy(rows_v.at[pl.ds(0, n)], psum_hbm.at[pl.ds(bh + off, n)])

    return k


def _phase_counts(D, E_pad):
    db = D // L
    nch = E_pad // (NW * C)
    epw = nch * C

    @functools.partial(
        pl.kernel,
        mesh=_mesh(),
        out_type=jax.ShapeDtypeStruct((NC * NP, D), jnp.float32),
        compiler_params=pltpu.CompilerParams(needs_layout_passes=False),
        scratch_types=[
            pltpu.VMEM((C,), jnp.int32),
            pltpu.VMEM((C, D), jnp.float32),
            pltpu.VMEM_SHARED((NP, D), jnp.float32),
        ],
    )
    def k(seg_hbm, pcnt_hbm, seg_v, buf_v, cnt_sh):
        cid = lax.axis_index("c")
        sid = lax.axis_index("s")
        w = sid * NC + cid

        zeros16 = jnp.zeros((L,), jnp.float32)
        ones16 = jnp.ones((L,), jnp.float32)

        @pl.loop(0, C)
        def _(r):
            for j in range(db):
                buf_v[r, pl.ds(j * L, L)] = zeros16

        bz = pl.multiple_of(sid * (NP // NS), 8)
        for off, n in _DCH:
            pltpu.sync_copy(buf_v.at[pl.ds(0, n)], cnt_sh.at[pl.ds(bz + off, n)])
        plsc.subcore_barrier()

        @pl.loop(0, C)
        def _(r):
            buf_v[r, pl.ds(0, L)] = ones16

        clamp16 = jnp.full((L,), CLAMP, jnp.int32)

        @pl.loop(0, nch)
        def _(i):
            b = pl.multiple_of(w * epw + i * C, C)
            pltpu.sync_copy(seg_hbm.at[pl.ds(b, C)], seg_v)
            for j in range(C // L):
                s = pl.ds(j * L, L)
                seg_v[s] = jnp.minimum(seg_v[s], clamp16)
            pltpu.sync_copy(buf_v, cnt_sh.at[seg_v], add=True)

        plsc.subcore_barrier()

        bh = pl.multiple_of(cid * NP + sid * (NP // NS), 8)
        for off, n in _DCH:
            pltpu.sync_copy(cnt_sh.at[pl.ds(bz + off, n)], buf_v.at[pl.ds(0, n)])
            pltpu.sync_copy(buf_v.at[pl.ds(0, n)], pcnt_hbm.at[pl.ds(bh + off, n)])

    return k


def _phase_merge(D):
    db = D // L
    nchunk = NP // C

    @functools.partial(
        pl.kernel,
        mesh=_mesh(),
        out_type=jax.ShapeDtypeStruct((NP, D), jnp.float32),
        compiler_params=pltpu.CompilerParams(needs_layout_passes=False),
        scratch_types=[
            pltpu.VMEM((C, D), jnp.float32),
            pltpu.VMEM((C, D), jnp.float32),
            pltpu.VMEM((C, D), jnp.float32),
            pltpu.VMEM((C, D), jnp.float32),
        ],
    )
    def k(psum_hbm, pcnt_hbm, out_hbm, p0_v, p1_v, c0_v, c1_v):
        cid = lax.axis_index("c")
        sid = lax.axis_index("s")
        w = sid * NC + cid

        for j in range((nchunk + NW - 1) // NW):
            ch = w + j * NW

            @pl.when(ch < nchunk)
            def _():
                base = pl.multiple_of(ch * C, 8)
                pltpu.sync_copy(psum_hbm.at[pl.ds(base, C)], p0_v)
                pltpu.sync_copy(psum_hbm.at[pl.ds(NP + base, C)], p1_v)
                pltpu.sync_copy(pcnt_hbm.at[pl.ds(base, C)], c0_v)
                pltpu.sync_copy(pcnt_hbm.at[pl.ds(NP + base, C)], c1_v)

                @pl.loop(0, C)
                def _(r):
                    cv = c0_v[r, pl.ds(0, L)] + c1_v[r, pl.ds(0, L)]
                    inv = jnp.where(cv > 0.0, 1.0 / jnp.maximum(cv, 1.0), 0.0)
                    for jj in range(db):
                        s = pl.ds(jj * L, L)
                        p0_v[r, s] = (p0_v[r, s] + p1_v[r, s]) * inv

                pltpu.sync_copy(p0_v, out_hbm.at[pl.ds(base, C)])

    return k


def kernel(features, neighbor_idx, segment_ids, num_neighbors):
    N, D = features.shape
    E = neighbor_idx.shape[0]
    E_pad = -(-E // (NW * C)) * (NW * C)
    pad = E_pad - E

    nbr_p = jnp.concatenate([neighbor_idx, jnp.zeros((pad,), jnp.int32)])
    seg_p = jnp.concatenate([segment_ids, jnp.full((pad,), BIG, jnp.int32)])

    psum = _phase_sums(N, D, E_pad)(features, nbr_p, seg_p)
    pcnt = _phase_counts(D, E_pad)(seg_p)
    out = _phase_merge(D)(psum, pcnt)
    return out[:N]
